# baseline (speedup 1.0000x reference)
import jax
import jax.numpy as jnp
from jax import lax
from jax.experimental import pallas as pl
from jax.experimental.pallas import tpu as pltpu
from jax.experimental.pallas import tpu_sc as plsc

N_NODES = 10000
N_HE = 10000
N_EDGES = 160000
D = 256

SEG_PAD = 10240
ROWS_PER_SUB = 640

NSUB = 16
CHUNK = 125
NCHUNK = 40
NCHUNK_CNT = 80

RBLK = 2000

_MESH = plsc.VectorSubcoreMesh(core_axis_name="c", subcore_axis_name="s")
_SC_PARAMS = pltpu.CompilerParams(use_tc_tiling_on_sc=False)


def _sc_stage(src_b, gtab, stab, zeros_rows):
    def body(src_hbm, gtab_hbm, stab_hbm, zero_hbm, out_hbm,
             acc, gt_v, st_v, r0, r1, s0, s1):
        c = lax.axis_index("c")
        s = lax.axis_index("s")
        pltpu.sync_copy(gtab_hbm.at[c, s], gt_v)
        pltpu.sync_copy(stab_hbm.at[c, s], st_v)
        pltpu.async_copy(src_hbm.at[gt_v.at[0]], r0, s0)
        pltpu.async_copy(src_hbm.at[gt_v.at[1]], r1, s1)
        pltpu.sync_copy(zero_hbm, acc.at[pl.ds(s * ROWS_PER_SUB, ROWS_PER_SUB)])
        plsc.subcore_barrier()

        last = NCHUNK // 2 - 1

        def step(t, carry):
            j0 = 2 * t
            j1 = 2 * t + 1
            pltpu.make_async_copy(src_hbm.at[gt_v.at[j0]], r0, s0).wait()
            pltpu.sync_copy(r0, acc.at[st_v.at[j0]], add=True)

            @pl.when(t < last)
            def _():
                pltpu.async_copy(src_hbm.at[gt_v.at[j0 + 2]], r0, s0)

            pltpu.make_async_copy(src_hbm.at[gt_v.at[j1]], r1, s1).wait()
            pltpu.sync_copy(r1, acc.at[st_v.at[j1]], add=True)

            @pl.when(t < last)
            def _():
                pltpu.async_copy(src_hbm.at[gt_v.at[j1 + 2]], r1, s1)

            return carry

        lax.fori_loop(0, NCHUNK // 2, step, 0)
        plsc.subcore_barrier()
        pltpu.sync_copy(acc.at[pl.ds(s * ROWS_PER_SUB, ROWS_PER_SUB)],
                        out_hbm.at[c, pl.ds(s * ROWS_PER_SUB, ROWS_PER_SUB)])

    return pl.kernel(
        body,
        out_type=jax.ShapeDtypeStruct((2, SEG_PAD, D), jnp.bfloat16),
        mesh=_MESH,
        compiler_params=_SC_PARAMS,
        scratch_types=[
            pltpu.VMEM_SHARED((SEG_PAD, D), jnp.bfloat16),
            pltpu.VMEM((NCHUNK, CHUNK), jnp.int32),
            pltpu.VMEM((NCHUNK, CHUNK), jnp.int32),
            pltpu.VMEM((CHUNK, D), jnp.bfloat16),
            pltpu.VMEM((CHUNK, D), jnp.bfloat16),
            pltpu.SemaphoreType.DMA,
            pltpu.SemaphoreType.DMA,
        ],
    )(src_b, gtab, stab, zeros_rows)


def _sc_counts(ctab, ones_rows, zeros_rows):
    def body(ctab_hbm, ones_hbm, zero_hbm, out_hbm, acc, ct_v, ones_v):
        c = lax.axis_index("c")
        s = lax.axis_index("s")
        pltpu.sync_copy(zero_hbm, acc.at[pl.ds(s * ROWS_PER_SUB, ROWS_PER_SUB)])
        pltpu.sync_copy(ctab_hbm.at[c, s], ct_v)
        pltpu.sync_copy(ones_hbm, ones_v)
        plsc.subcore_barrier()

        def step(j, carry):
            pltpu.sync_copy(ones_v, acc.at[ct_v.at[j]], add=True)
            return carry

        lax.fori_loop(0, NCHUNK_CNT, step, 0)
        plsc.subcore_barrier()
        pltpu.sync_copy(acc.at[pl.ds(s * ROWS_PER_SUB, ROWS_PER_SUB)],
                        out_hbm.at[c, pl.ds(s * ROWS_PER_SUB, ROWS_PER_SUB)])

    return pl.kernel(
        body,
        out_type=jax.ShapeDtypeStruct((2, SEG_PAD, 16), jnp.float32),
        mesh=_MESH,
        compiler_params=_SC_PARAMS,
        scratch_types=[
            pltpu.VMEM_SHARED((SEG_PAD, 16), jnp.float32),
            pltpu.VMEM((NCHUNK_CNT, CHUNK), jnp.int32),
            pltpu.VMEM((CHUNK, 16), jnp.float32),
        ],
    )(ctab, ones_rows, zeros_rows)


def _mm2d(x, W, b):
    def kern(x_ref, w_ref, b_ref, o_ref, ob_ref):
        acc = (jnp.dot(x_ref[...], w_ref[...],
                       preferred_element_type=jnp.float32) + b_ref[0])
        o_ref[...] = acc
        ob_ref[...] = acc.astype(jnp.bfloat16).reshape(2 * RBLK, D // 2)

    ospec = pl.BlockSpec((RBLK, D), lambda r: (r, 0))
    return pl.pallas_call(
        kern,
        grid=(N_NODES // RBLK,),
        in_specs=[ospec,
                  pl.BlockSpec((D, D), lambda r: (0, 0)),
                  pl.BlockSpec((1, D), lambda r: (0, 0))],
        out_specs=[ospec, pl.BlockSpec((2 * RBLK, D // 2), lambda r: (r, 0))],
        out_shape=[jax.ShapeDtypeStruct((N_NODES, D), jnp.float32),
                   jax.ShapeDtypeStruct((2 * N_NODES, D // 2), jnp.bfloat16)],
    )(x, W, b)


def _mm3d(parts, cnt, hw, W, b):
    def kern(p_ref, cnt_ref, hw_ref, w_ref, b_ref, o_ref):
        rcp = 1.0 / (cnt_ref[...] + 1e-8)
        hs = (p_ref[0].astype(jnp.float32).reshape(RBLK, D)
              + p_ref[1].astype(jnp.float32).reshape(RBLK, D)) * rcp
        acc = (jnp.dot(hs, w_ref[...], preferred_element_type=jnp.float32)
               + b_ref[0])
        o_ref[...] = (acc * hw_ref[...]).astype(jnp.bfloat16).reshape(
            2 * RBLK, D // 2)

    return pl.pallas_call(
        kern,
        grid=(N_NODES // RBLK,),
        in_specs=[pl.BlockSpec((2, 2 * RBLK, D // 2), lambda r: (0, r, 0)),
                  pl.BlockSpec((RBLK, 1), lambda r: (r, 0)),
                  pl.BlockSpec((RBLK, 1), lambda r: (r, 0)),
                  pl.BlockSpec((D, D), lambda r: (0, 0)),
                  pl.BlockSpec((1, D), lambda r: (0, 0))],
        out_specs=pl.BlockSpec((2 * RBLK, D // 2), lambda r: (r, 0)),
        out_shape=jax.ShapeDtypeStruct((2 * N_HE, D // 2), jnp.bfloat16),
    )(parts, cnt, hw, W, b)


def _segment_mean_norm(p_ref, cnt_ref, xt_ref, g_ref, b_ref):
    rcp = 1.0 / jnp.maximum(cnt_ref[...], 1.0)
    t = (p_ref[0].astype(jnp.float32).reshape(RBLK, D)
         + p_ref[1].astype(jnp.float32).reshape(RBLK, D)) * rcp + xt_ref[...]
    m = jnp.mean(t, axis=-1, keepdims=True)
    d = t - m
    var = jnp.mean(d * d, axis=-1, keepdims=True)
    y = d * lax.rsqrt(var + 1e-5) * g_ref[0] + b_ref[0]
    return jnp.where(y >= 0, y, 0.2 * y)


def _norm_mm(parts, cnt, xt, g, b, W, bn):
    def kern(p_ref, cnt_ref, xt_ref, g_ref, b_ref, w_ref, bn_ref,
             h_ref, o_ref, ob_ref):
        y = _segment_mean_norm(p_ref, cnt_ref, xt_ref, g_ref, b_ref)
        h_ref[...] = y
        acc = (jnp.dot(y, w_ref[...], preferred_element_type=jnp.float32)
               + bn_ref[0])
        o_ref[...] = acc
        ob_ref[...] = acc.astype(jnp.bfloat16).reshape(2 * RBLK, D // 2)

    ospec = pl.BlockSpec((RBLK, D), lambda r: (r, 0))
    return pl.pallas_call(
        kern,
        grid=(N_NODES // RBLK,),
        in_specs=[pl.BlockSpec((2, 2 * RBLK, D // 2), lambda r: (0, r, 0)),
                  pl.BlockSpec((RBLK, 1), lambda r: (r, 0)),
                  ospec,
                  pl.BlockSpec((1, D), lambda r: (0, 0)),
                  pl.BlockSpec((1, D), lambda r: (0, 0)),
                  pl.BlockSpec((D, D), lambda r: (0, 0)),
                  pl.BlockSpec((1, D), lambda r: (0, 0))],
        out_specs=[ospec, ospec,
                   pl.BlockSpec((2 * RBLK, D // 2), lambda r: (r, 0))],
        out_shape=[jax.ShapeDtypeStruct((N_NODES, D), jnp.float32),
                   jax.ShapeDtypeStruct((N_NODES, D), jnp.float32),
                   jax.ShapeDtypeStruct((2 * N_NODES, D // 2), jnp.bfloat16)],
    )(parts, cnt, xt, g, b, W, bn)


def _norm_final(parts, cnt, xt, g, b, resid):
    def kern(p_ref, cnt_ref, xt_ref, g_ref, b_ref, res_ref, o_ref):
        y = _segment_mean_norm(p_ref, cnt_ref, xt_ref, g_ref, b_ref)
        o_ref[...] = y + res_ref[...]

    ospec = pl.BlockSpec((RBLK, D), lambda r: (r, 0))
    return pl.pallas_call(
        kern,
        grid=(N_NODES // RBLK,),
        in_specs=[pl.BlockSpec((2, 2 * RBLK, D // 2), lambda r: (0, r, 0)),
                  pl.BlockSpec((RBLK, 1), lambda r: (r, 0)),
                  ospec,
                  pl.BlockSpec((1, D), lambda r: (0, 0)),
                  pl.BlockSpec((1, D), lambda r: (0, 0)),
                  ospec],
        out_specs=ospec,
        out_shape=jax.ShapeDtypeStruct((N_NODES, D), jnp.float32),
    )(parts, cnt, xt, g, b, resid)


def kernel(x, hyperedge_index, hyperedge_weight,
           Wn1, bn1, Wh1, bh1, g1, be1,
           Wn2, bn2, Wh2, bh2, g2, be2):
    node_idx = hyperedge_index[0]
    he_idx = hyperedge_index[1]

    stage_shape = (2, NSUB, NCHUNK, CHUNK)
    gtabA = node_idx.reshape(stage_shape)
    stabA = he_idx.reshape(stage_shape)
    gtabB = he_idx.reshape(stage_shape)
    stabB = node_idx.reshape(stage_shape)
    ctab = jnp.stack([he_idx, node_idx]).reshape(2, NSUB, NCHUNK_CNT, CHUNK)

    zerosD = jnp.zeros((ROWS_PER_SUB, D), jnp.bfloat16)
    zeros16 = jnp.zeros((ROWS_PER_SUB, 16), jnp.float32)
    ones16 = jnp.ones((CHUNK, 16), jnp.float32)

    cnts = _sc_counts(ctab, ones16, zeros16)
    he_cnt = cnts[0, :N_HE, 0:1]
    n_cnt = cnts[1, :N_NODES, 0:1]
    hw = hyperedge_weight.reshape(N_HE, 1)

    def as_rows(b):
        return b.reshape(-1, D)

    def as_lanes(p):
        return p.reshape(2, 2 * SEG_PAD, D // 2)

    xt1, xt1_b = _mm2d(x, Wn1, bn1.reshape(1, D))
    he_p1 = _sc_stage(as_rows(xt1_b), gtabA, stabA, zerosD)
    hew1 = _mm3d(as_lanes(he_p1), he_cnt, hw, Wh1, bh1.reshape(1, D))
    n_p1 = _sc_stage(as_rows(hew1), gtabB, stabB, zerosD)
    h1, xt2, xt2_b = _norm_mm(as_lanes(n_p1), n_cnt, xt1, g1.reshape(1, D),
                              be1.reshape(1, D), Wn2, bn2.reshape(1, D))

    he_p2 = _sc_stage(as_rows(xt2_b), gtabA, stabA, zerosD)
    hew2 = _mm3d(as_lanes(he_p2), he_cnt, hw, Wh2, bh2.reshape(1, D))
    n_p2 = _sc_stage(as_rows(hew2), gtabB, stabB, zerosD)
    return _norm_final(as_lanes(n_p2), n_cnt, xt2, g2.reshape(1, D),
                       be2.reshape(1, D), h1)


# speedup vs baseline: 1.1724x; 1.1724x over previous
import jax
import jax.numpy as jnp
from jax import lax
from jax.experimental import pallas as pl
from jax.experimental.pallas import tpu as pltpu
from jax.experimental.pallas import tpu_sc as plsc

N_NODES = 10000
N_HE = 10000
N_EDGES = 160000
D = 256

SEG_PAD = 10240
ROWS_PER_SUB = 640

NSUB = 16
CHUNK = 125
NCHUNK = 40
NCHUNK_CNT = 80

RBLK = 2000

_MESH = plsc.VectorSubcoreMesh(core_axis_name="c", subcore_axis_name="s")
_SC_PARAMS = pltpu.CompilerParams(use_tc_tiling_on_sc=False)


def _sc_stage(src_b, gtab, stab, zeros_rows):
    def body(src_hbm, gtab_hbm, stab_hbm, zero_hbm, out_hbm,
             acc, gt_v, st_v, r0, r1, s0, s1):
        c = lax.axis_index("c")
        s = lax.axis_index("s")
        pltpu.sync_copy(gtab_hbm.at[c, s], gt_v)
        pltpu.sync_copy(stab_hbm.at[c, s], st_v)
        pltpu.async_copy(src_hbm.at[gt_v.at[0]], r0, s0)
        pltpu.async_copy(src_hbm.at[gt_v.at[1]], r1, s1)
        pltpu.sync_copy(zero_hbm, acc.at[pl.ds(s * ROWS_PER_SUB, ROWS_PER_SUB)])
        plsc.subcore_barrier()

        last = NCHUNK // 2 - 1

        def step(t, carry):
            j0 = 2 * t
            j1 = 2 * t + 1
            pltpu.make_async_copy(src_hbm.at[gt_v.at[j0]], r0, s0).wait()
            pltpu.sync_copy(r0, acc.at[st_v.at[j0]], add=True)

            @pl.when(t < last)
            def _():
                pltpu.async_copy(src_hbm.at[gt_v.at[j0 + 2]], r0, s0)

            pltpu.make_async_copy(src_hbm.at[gt_v.at[j1]], r1, s1).wait()
            pltpu.sync_copy(r1, acc.at[st_v.at[j1]], add=True)

            @pl.when(t < last)
            def _():
                pltpu.async_copy(src_hbm.at[gt_v.at[j1 + 2]], r1, s1)

            return carry

        lax.fori_loop(0, NCHUNK // 2, step, 0)
        plsc.subcore_barrier()
        pltpu.sync_copy(acc.at[pl.ds(s * ROWS_PER_SUB, ROWS_PER_SUB)],
                        out_hbm.at[c, pl.ds(s * ROWS_PER_SUB, ROWS_PER_SUB)])

    return pl.kernel(
        body,
        out_type=jax.ShapeDtypeStruct((2, SEG_PAD, D), jnp.bfloat16),
        mesh=_MESH,
        compiler_params=_SC_PARAMS,
        scratch_types=[
            pltpu.VMEM_SHARED((SEG_PAD, D), jnp.bfloat16),
            pltpu.VMEM((NCHUNK, CHUNK), jnp.int32),
            pltpu.VMEM((NCHUNK, CHUNK), jnp.int32),
            pltpu.VMEM((CHUNK, D), jnp.bfloat16),
            pltpu.VMEM((CHUNK, D), jnp.bfloat16),
            pltpu.SemaphoreType.DMA,
            pltpu.SemaphoreType.DMA,
        ],
    )(src_b, gtab, stab, zeros_rows)


def _sc_counts(ctab, ones_rows, zeros_rows):
    def body(ctab_hbm, ones_hbm, zero_hbm, out_hbm, acc, ct_v, ones_v):
        c = lax.axis_index("c")
        s = lax.axis_index("s")
        pltpu.sync_copy(zero_hbm, acc.at[pl.ds(s * ROWS_PER_SUB, ROWS_PER_SUB)])
        pltpu.sync_copy(ctab_hbm.at[c, s], ct_v)
        pltpu.sync_copy(ones_hbm, ones_v)
        plsc.subcore_barrier()

        def step(j, carry):
            pltpu.sync_copy(ones_v, acc.at[ct_v.at[j]], add=True)
            return carry

        lax.fori_loop(0, NCHUNK_CNT, step, 0)
        plsc.subcore_barrier()
        pltpu.sync_copy(acc.at[pl.ds(s * ROWS_PER_SUB, ROWS_PER_SUB)],
                        out_hbm.at[c, pl.ds(s * ROWS_PER_SUB, ROWS_PER_SUB)])

    return pl.kernel(
        body,
        out_type=jax.ShapeDtypeStruct((2, SEG_PAD, 16), jnp.float32),
        mesh=_MESH,
        compiler_params=_SC_PARAMS,
        scratch_types=[
            pltpu.VMEM_SHARED((SEG_PAD, 16), jnp.float32),
            pltpu.VMEM((NCHUNK_CNT, CHUNK), jnp.int32),
            pltpu.VMEM((CHUNK, 16), jnp.float32),
        ],
    )(ctab, ones_rows, zeros_rows)


def _mm2d(x, W, b):
    def kern(x_ref, w_ref, b_ref, o_ref, ob_ref):
        acc = (jnp.dot(x_ref[...], w_ref[...],
                       preferred_element_type=jnp.float32) + b_ref[0])
        o_ref[...] = acc
        ob_ref[...] = acc.astype(jnp.bfloat16)

    ospec = pl.BlockSpec((RBLK, D), lambda r: (r, 0))
    return pl.pallas_call(
        kern,
        grid=(N_NODES // RBLK,),
        in_specs=[ospec,
                  pl.BlockSpec((D, D), lambda r: (0, 0)),
                  pl.BlockSpec((1, D), lambda r: (0, 0))],
        out_specs=[ospec, ospec],
        out_shape=[jax.ShapeDtypeStruct((N_NODES, D), jnp.float32),
                   jax.ShapeDtypeStruct((N_NODES, D), jnp.bfloat16)],
    )(x, W, b)


def _mm3d(parts, cnt, hw, W, b):
    def kern(p_ref, cnt_ref, hw_ref, w_ref, b_ref, o_ref):
        rcp = 1.0 / (cnt_ref[0, :, 0:1] + 1e-8)
        hs = (p_ref[0].astype(jnp.float32)
              + p_ref[1].astype(jnp.float32)) * rcp
        acc = (jnp.dot(hs, w_ref[...], preferred_element_type=jnp.float32)
               + b_ref[0])
        o_ref[...] = (acc * hw_ref[...]).astype(jnp.bfloat16)

    return pl.pallas_call(
        kern,
        grid=(N_NODES // RBLK,),
        in_specs=[pl.BlockSpec((2, RBLK, D), lambda r: (0, r, 0)),
                  pl.BlockSpec((1, RBLK, 16), lambda r: (0, r, 0)),
                  pl.BlockSpec((RBLK, 1), lambda r: (r, 0)),
                  pl.BlockSpec((D, D), lambda r: (0, 0)),
                  pl.BlockSpec((1, D), lambda r: (0, 0))],
        out_specs=pl.BlockSpec((RBLK, D), lambda r: (r, 0)),
        out_shape=jax.ShapeDtypeStruct((N_HE, D), jnp.bfloat16),
    )(parts, cnt, hw, W, b)


def _segment_mean_norm(p_ref, cnt_ref, xt_ref, g_ref, b_ref):
    rcp = 1.0 / jnp.maximum(cnt_ref[0, :, 0:1], 1.0)
    t = (p_ref[0].astype(jnp.float32)
         + p_ref[1].astype(jnp.float32)) * rcp + xt_ref[...]
    m = jnp.mean(t, axis=-1, keepdims=True)
    d = t - m
    var = jnp.mean(d * d, axis=-1, keepdims=True)
    y = d * lax.rsqrt(var + 1e-5) * g_ref[0] + b_ref[0]
    return jnp.where(y >= 0, y, 0.2 * y)


def _norm_mm(parts, cnt, xt, g, b, W, bn):
    def kern(p_ref, cnt_ref, xt_ref, g_ref, b_ref, w_ref, bn_ref,
             h_ref, o_ref, ob_ref):
        y = _segment_mean_norm(p_ref, cnt_ref, xt_ref, g_ref, b_ref)
        h_ref[...] = y
        acc = (jnp.dot(y, w_ref[...], preferred_element_type=jnp.float32)
               + bn_ref[0])
        o_ref[...] = acc
        ob_ref[...] = acc.astype(jnp.bfloat16)

    ospec = pl.BlockSpec((RBLK, D), lambda r: (r, 0))
    return pl.pallas_call(
        kern,
        grid=(N_NODES // RBLK,),
        in_specs=[pl.BlockSpec((2, RBLK, D), lambda r: (0, r, 0)),
                  pl.BlockSpec((1, RBLK, 16), lambda r: (1, r, 0)),
                  ospec,
                  pl.BlockSpec((1, D), lambda r: (0, 0)),
                  pl.BlockSpec((1, D), lambda r: (0, 0)),
                  pl.BlockSpec((D, D), lambda r: (0, 0)),
                  pl.BlockSpec((1, D), lambda r: (0, 0))],
        out_specs=[ospec, ospec, ospec],
        out_shape=[jax.ShapeDtypeStruct((N_NODES, D), jnp.float32),
                   jax.ShapeDtypeStruct((N_NODES, D), jnp.float32),
                   jax.ShapeDtypeStruct((N_NODES, D), jnp.bfloat16)],
    )(parts, cnt, xt, g, b, W, bn)


def _norm_final(parts, cnt, xt, g, b, resid):
    def kern(p_ref, cnt_ref, xt_ref, g_ref, b_ref, res_ref, o_ref):
        y = _segment_mean_norm(p_ref, cnt_ref, xt_ref, g_ref, b_ref)
        o_ref[...] = y + res_ref[...]

    ospec = pl.BlockSpec((RBLK, D), lambda r: (r, 0))
    return pl.pallas_call(
        kern,
        grid=(N_NODES // RBLK,),
        in_specs=[pl.BlockSpec((2, RBLK, D), lambda r: (0, r, 0)),
                  pl.BlockSpec((1, RBLK, 16), lambda r: (1, r, 0)),
                  ospec,
                  pl.BlockSpec((1, D), lambda r: (0, 0)),
                  pl.BlockSpec((1, D), lambda r: (0, 0)),
                  ospec],
        out_specs=ospec,
        out_shape=jax.ShapeDtypeStruct((N_NODES, D), jnp.float32),
    )(parts, cnt, xt, g, b, resid)


def kernel(x, hyperedge_index, hyperedge_weight,
           Wn1, bn1, Wh1, bh1, g1, be1,
           Wn2, bn2, Wh2, bh2, g2, be2):
    node_idx = hyperedge_index[0]
    he_idx = hyperedge_index[1]

    stage_shape = (2, NSUB, NCHUNK, CHUNK)
    gtabA = node_idx.reshape(stage_shape)
    stabA = he_idx.reshape(stage_shape)
    gtabB = he_idx.reshape(stage_shape)
    stabB = node_idx.reshape(stage_shape)
    ctab = jnp.stack([he_idx, node_idx]).reshape(2, NSUB, NCHUNK_CNT, CHUNK)

    zerosD = jnp.zeros((ROWS_PER_SUB, D), jnp.bfloat16)
    zeros16 = jnp.zeros((ROWS_PER_SUB, 16), jnp.float32)
    ones16 = jnp.ones((CHUNK, 16), jnp.float32)

    cnts = _sc_counts(ctab, ones16, zeros16)
    hw = hyperedge_weight.reshape(N_HE, 1)

    xt1, xt1_b = _mm2d(x, Wn1, bn1.reshape(1, D))
    he_p1 = _sc_stage(xt1_b, gtabA, stabA, zerosD)
    hew1 = _mm3d(he_p1, cnts, hw, Wh1, bh1.reshape(1, D))
    n_p1 = _sc_stage(hew1, gtabB, stabB, zerosD)
    h1, xt2, xt2_b = _norm_mm(n_p1, cnts, xt1, g1.reshape(1, D),
                              be1.reshape(1, D), Wn2, bn2.reshape(1, D))

    he_p2 = _sc_stage(xt2_b, gtabA, stabA, zerosD)
    hew2 = _mm3d(he_p2, cnts, hw, Wh2, bh2.reshape(1, D))
    n_p2 = _sc_stage(hew2, gtabB, stabB, zerosD)
    return _norm_final(n_p2, cnts, xt2, g2.reshape(1, D), be2.reshape(1, D),
                       h1)


# speedup vs baseline: 1.1989x; 1.0226x over previous
import jax
import jax.numpy as jnp
from jax import lax
from jax.experimental import pallas as pl
from jax.experimental.pallas import tpu as pltpu
from jax.experimental.pallas import tpu_sc as plsc

N_NODES = 10000
N_HE = 10000
N_EDGES = 160000
D = 256

SEG_PAD = 10240
ROWS_PER_SUB = 640

NSUB = 16
CHUNK = 50
NCHUNK = 100
NCHUNK_CNT = 200

RBLK = 2000

_MESH = plsc.VectorSubcoreMesh(core_axis_name="c", subcore_axis_name="s")
_SC_PARAMS = pltpu.CompilerParams(use_tc_tiling_on_sc=False)


def _sc_stage(src_b, gtab, stab, zeros_rows):
    def body(src_hbm, gtab_hbm, stab_hbm, zero_hbm, out_hbm,
             acc, gt_v, st_v, r0, r1, r2, r3, s0, s1, s2, s3):
        c = lax.axis_index("c")
        s = lax.axis_index("s")
        pltpu.sync_copy(gtab_hbm.at[c, s], gt_v)
        pltpu.sync_copy(stab_hbm.at[c, s], st_v)
        rows = (r0, r1, r2, r3)
        sems = (s0, s1, s2, s3)
        for k in range(4):
            pltpu.async_copy(src_hbm.at[gt_v.at[k]], rows[k], sems[k])
        pltpu.sync_copy(zero_hbm, acc.at[pl.ds(s * ROWS_PER_SUB, ROWS_PER_SUB)])
        plsc.subcore_barrier()

        last = NCHUNK // 4 - 1

        def step(t, carry):
            base = 4 * t
            for k in range(4):
                j = base + k
                pltpu.make_async_copy(src_hbm.at[gt_v.at[j]],
                                      rows[k], sems[k]).wait()
                pltpu.sync_copy(rows[k], acc.at[st_v.at[j]], add=True)

                @pl.when(t < last)
                def _():
                    pltpu.async_copy(src_hbm.at[gt_v.at[j + 4]],
                                     rows[k], sems[k])

            return carry

        lax.fori_loop(0, NCHUNK // 4, step, 0)
        plsc.subcore_barrier()
        pltpu.sync_copy(acc.at[pl.ds(s * ROWS_PER_SUB, ROWS_PER_SUB)],
                        out_hbm.at[c, pl.ds(s * ROWS_PER_SUB, ROWS_PER_SUB)])

    return pl.kernel(
        body,
        out_type=jax.ShapeDtypeStruct((2, SEG_PAD, D), jnp.bfloat16),
        mesh=_MESH,
        compiler_params=_SC_PARAMS,
        scratch_types=[
            pltpu.VMEM_SHARED((SEG_PAD, D), jnp.bfloat16),
            pltpu.VMEM((NCHUNK, CHUNK), jnp.int32),
            pltpu.VMEM((NCHUNK, CHUNK), jnp.int32),
            pltpu.VMEM((CHUNK, D), jnp.bfloat16),
            pltpu.VMEM((CHUNK, D), jnp.bfloat16),
            pltpu.VMEM((CHUNK, D), jnp.bfloat16),
            pltpu.VMEM((CHUNK, D), jnp.bfloat16),
            pltpu.SemaphoreType.DMA,
            pltpu.SemaphoreType.DMA,
            pltpu.SemaphoreType.DMA,
            pltpu.SemaphoreType.DMA,
        ],
    )(src_b, gtab, stab, zeros_rows)


def _sc_counts(ctab, ones_rows, zeros_rows):
    def body(ctab_hbm, ones_hbm, zero_hbm, out_hbm, acc, ct_v, ones_v):
        c = lax.axis_index("c")
        s = lax.axis_index("s")
        pltpu.sync_copy(zero_hbm, acc.at[pl.ds(s * ROWS_PER_SUB, ROWS_PER_SUB)])
        pltpu.sync_copy(ctab_hbm.at[c, s], ct_v)
        pltpu.sync_copy(ones_hbm, ones_v)
        plsc.subcore_barrier()

        def step(j, carry):
            pltpu.sync_copy(ones_v, acc.at[ct_v.at[j]], add=True)
            return carry

        lax.fori_loop(0, NCHUNK_CNT, step, 0)
        plsc.subcore_barrier()
        pltpu.sync_copy(acc.at[pl.ds(s * ROWS_PER_SUB, ROWS_PER_SUB)],
                        out_hbm.at[c, pl.ds(s * ROWS_PER_SUB, ROWS_PER_SUB)])

    return pl.kernel(
        body,
        out_type=jax.ShapeDtypeStruct((2, SEG_PAD, 16), jnp.float32),
        mesh=_MESH,
        compiler_params=_SC_PARAMS,
        scratch_types=[
            pltpu.VMEM_SHARED((SEG_PAD, 16), jnp.float32),
            pltpu.VMEM((NCHUNK_CNT, CHUNK), jnp.int32),
            pltpu.VMEM((CHUNK, 16), jnp.float32),
        ],
    )(ctab, ones_rows, zeros_rows)


def _mm2d(x, W, b):
    def kern(x_ref, w_ref, b_ref, o_ref, ob_ref):
        acc = (jnp.dot(x_ref[...], w_ref[...],
                       preferred_element_type=jnp.float32) + b_ref[0])
        o_ref[...] = acc
        ob_ref[...] = acc.astype(jnp.bfloat16)

    ospec = pl.BlockSpec((RBLK, D), lambda r: (r, 0))
    return pl.pallas_call(
        kern,
        grid=(N_NODES // RBLK,),
        in_specs=[ospec,
                  pl.BlockSpec((D, D), lambda r: (0, 0)),
                  pl.BlockSpec((1, D), lambda r: (0, 0))],
        out_specs=[ospec, ospec],
        out_shape=[jax.ShapeDtypeStruct((N_NODES, D), jnp.float32),
                   jax.ShapeDtypeStruct((N_NODES, D), jnp.bfloat16)],
    )(x, W, b)


def _mm3d(parts, cnt, hw, W, b):
    def kern(p_ref, cnt_ref, hw_ref, w_ref, b_ref, o_ref):
        rcp = 1.0 / (cnt_ref[0, :, 0:1] + 1e-8)
        hs = (p_ref[0].astype(jnp.float32)
              + p_ref[1].astype(jnp.float32)) * rcp
        acc = (jnp.dot(hs, w_ref[...], preferred_element_type=jnp.float32)
               + b_ref[0])
        o_ref[...] = (acc * hw_ref[...]).astype(jnp.bfloat16)

    return pl.pallas_call(
        kern,
        grid=(N_NODES // RBLK,),
        in_specs=[pl.BlockSpec((2, RBLK, D), lambda r: (0, r, 0)),
                  pl.BlockSpec((1, RBLK, 16), lambda r: (0, r, 0)),
                  pl.BlockSpec((RBLK, 1), lambda r: (r, 0)),
                  pl.BlockSpec((D, D), lambda r: (0, 0)),
                  pl.BlockSpec((1, D), lambda r: (0, 0))],
        out_specs=pl.BlockSpec((RBLK, D), lambda r: (r, 0)),
        out_shape=jax.ShapeDtypeStruct((N_HE, D), jnp.bfloat16),
    )(parts, cnt, hw, W, b)


def _segment_mean_norm(p_ref, cnt_ref, xt_ref, g_ref, b_ref):
    rcp = 1.0 / jnp.maximum(cnt_ref[0, :, 0:1], 1.0)
    t = (p_ref[0].astype(jnp.float32)
         + p_ref[1].astype(jnp.float32)) * rcp + xt_ref[...]
    m = jnp.mean(t, axis=-1, keepdims=True)
    d = t - m
    var = jnp.mean(d * d, axis=-1, keepdims=True)
    y = d * lax.rsqrt(var + 1e-5) * g_ref[0] + b_ref[0]
    return jnp.where(y >= 0, y, 0.2 * y)


def _norm_mm(parts, cnt, xt, g, b, W, bn):
    def kern(p_ref, cnt_ref, xt_ref, g_ref, b_ref, w_ref, bn_ref,
             h_ref, o_ref, ob_ref):
        y = _segment_mean_norm(p_ref, cnt_ref, xt_ref, g_ref, b_ref)
        h_ref[...] = y
        acc = (jnp.dot(y, w_ref[...], preferred_element_type=jnp.float32)
               + bn_ref[0])
        o_ref[...] = acc
        ob_ref[...] = acc.astype(jnp.bfloat16)

    ospec = pl.BlockSpec((RBLK, D), lambda r: (r, 0))
    return pl.pallas_call(
        kern,
        grid=(N_NODES // RBLK,),
        in_specs=[pl.BlockSpec((2, RBLK, D), lambda r: (0, r, 0)),
                  pl.BlockSpec((1, RBLK, 16), lambda r: (1, r, 0)),
                  ospec,
                  pl.BlockSpec((1, D), lambda r: (0, 0)),
                  pl.BlockSpec((1, D), lambda r: (0, 0)),
                  pl.BlockSpec((D, D), lambda r: (0, 0)),
                  pl.BlockSpec((1, D), lambda r: (0, 0))],
        out_specs=[ospec, ospec, ospec],
        out_shape=[jax.ShapeDtypeStruct((N_NODES, D), jnp.float32),
                   jax.ShapeDtypeStruct((N_NODES, D), jnp.float32),
                   jax.ShapeDtypeStruct((N_NODES, D), jnp.bfloat16)],
    )(parts, cnt, xt, g, b, W, bn)


def _norm_final(parts, cnt, xt, g, b, resid):
    def kern(p_ref, cnt_ref, xt_ref, g_ref, b_ref, res_ref, o_ref):
        y = _segment_mean_norm(p_ref, cnt_ref, xt_ref, g_ref, b_ref)
        o_ref[...] = y + res_ref[...]

    ospec = pl.BlockSpec((RBLK, D), lambda r: (r, 0))
    return pl.pallas_call(
        kern,
        grid=(N_NODES // RBLK,),
        in_specs=[pl.BlockSpec((2, RBLK, D), lambda r: (0, r, 0)),
                  pl.BlockSpec((1, RBLK, 16), lambda r: (1, r, 0)),
                  ospec,
                  pl.BlockSpec((1, D), lambda r: (0, 0)),
                  pl.BlockSpec((1, D), lambda r: (0, 0)),
                  ospec],
        out_specs=ospec,
        out_shape=jax.ShapeDtypeStruct((N_NODES, D), jnp.float32),
    )(parts, cnt, xt, g, b, resid)


def kernel(x, hyperedge_index, hyperedge_weight,
           Wn1, bn1, Wh1, bh1, g1, be1,
           Wn2, bn2, Wh2, bh2, g2, be2):
    node_idx = hyperedge_index[0]
    he_idx = hyperedge_index[1]

    stage_shape = (2, NSUB, NCHUNK, CHUNK)
    gtabA = node_idx.reshape(stage_shape)
    stabA = he_idx.reshape(stage_shape)
    gtabB = he_idx.reshape(stage_shape)
    stabB = node_idx.reshape(stage_shape)
    ctab = jnp.stack([he_idx, node_idx]).reshape(2, NSUB, NCHUNK_CNT, CHUNK)

    zerosD = jnp.zeros((ROWS_PER_SUB, D), jnp.bfloat16)
    zeros16 = jnp.zeros((ROWS_PER_SUB, 16), jnp.float32)
    ones16 = jnp.ones((CHUNK, 16), jnp.float32)

    cnts = _sc_counts(ctab, ones16, zeros16)
    hw = hyperedge_weight.reshape(N_HE, 1)

    xt1, xt1_b = _mm2d(x, Wn1, bn1.reshape(1, D))
    he_p1 = _sc_stage(xt1_b, gtabA, stabA, zerosD)
    hew1 = _mm3d(he_p1, cnts, hw, Wh1, bh1.reshape(1, D))
    n_p1 = _sc_stage(hew1, gtabB, stabB, zerosD)
    h1, xt2, xt2_b = _norm_mm(n_p1, cnts, xt1, g1.reshape(1, D),
                              be1.reshape(1, D), Wn2, bn2.reshape(1, D))

    he_p2 = _sc_stage(xt2_b, gtabA, stabA, zerosD)
    hew2 = _mm3d(he_p2, cnts, hw, Wh2, bh2.reshape(1, D))
    n_p2 = _sc_stage(hew2, gtabB, stabB, zerosD)
    return _norm_final(n_p2, cnts, xt2, g2.reshape(1, D), be2.reshape(1, D),
                       h1)
